# baseline (speedup 1.0000x reference)
import dataclasses
import functools

import jax
import jax.numpy as jnp
from jax import lax
from jax.experimental import pallas as pl
from jax.experimental.pallas import tpu as pltpu
from jax.experimental.pallas import tpu_sc as plsc

_NUM_NODES = 100000
_D = 128
_B = 102400
_CTX = 10
_NPAIR = _CTX - 1
_EPS = 1e-15

_NW = 32
_WALKS = 2 * _B
_WPC = 16
_IDS_PER_CHUNK = _WPC * _CTX
_GRP = 40
_NGRP = _IDS_PER_CHUNK // _GRP
_CHUNKS = _WALKS // (_NW * _WPC)
_OUT_ROWS = _WPC * _NPAIR


def _xlane_gather(v, idx):
    dnums = lax.GatherDimensionNumbers(
        offset_dims=(), collapsed_slice_dims=(0,), start_index_map=(0,)
    )
    return lax.gather(
        v, idx[:, None], dnums, (1,),
        mode=lax.GatherScatterMode.PROMISE_IN_BOUNDS,
    )


def _sc_dots(embedding, ids3d):
    mesh = plsc.VectorSubcoreMesh(core_axis_name="c", subcore_axis_name="s")
    cp = pltpu.CompilerParams()
    if "needs_layout_passes" in pltpu.CompilerParams.__dataclass_fields__:
        cp = dataclasses.replace(cp, needs_layout_passes=False)

    @functools.partial(
        pl.kernel,
        out_type=jax.ShapeDtypeStruct((_WALKS * _NPAIR,), jnp.float32),
        mesh=mesh,
        compiler_params=cp,
        scratch_types=[
            pltpu.VMEM((_NGRP, _GRP), jnp.int32),
            pltpu.VMEM((_NGRP, _GRP), jnp.int32),
            pltpu.VMEM((_IDS_PER_CHUNK, _D), jnp.float32),
            pltpu.VMEM((_IDS_PER_CHUNK, _D), jnp.float32),
            pltpu.VMEM((_OUT_ROWS, 16), jnp.float32),
            pltpu.VMEM((_OUT_ROWS,), jnp.float32),
            pltpu.VMEM((_OUT_ROWS,), jnp.float32),
            pltpu.SemaphoreType.DMA,
            pltpu.SemaphoreType.DMA,
            pltpu.SemaphoreType.DMA,
            pltpu.SemaphoreType.DMA,
            pltpu.SemaphoreType.DMA,
            pltpu.SemaphoreType.DMA,
        ],
    )
    def sc_kernel(table_hbm, ids_hbm, out_hbm,
                  idx0, idx1, rows0, rows1, staged, ob0, ob1,
                  isem0, isem1, rsem0, rsem1, osem0, osem1):
        wid = lax.axis_index("s") * 2 + lax.axis_index("c")
        cc0 = wid * _CHUNKS

        idxb = (idx0, idx1)
        rowb = (rows0, rows1)
        outb = (ob0, ob1)
        isem = (isem0, isem1)
        rsem = (rsem0, rsem1)
        osem = (osem0, osem1)

        def start_gathers(b, _):
            for grp in range(_NGRP):
                pltpu.async_copy(
                    table_hbm.at[idxb[b].at[grp]],
                    rowb[b].at[pl.ds(grp * _GRP, _GRP)],
                    rsem[b],
                )

        def wait_gathers(b):
            for grp in range(_NGRP):
                pltpu.make_async_copy(
                    table_hbm.at[idxb[b].at[grp]],
                    rowb[b].at[pl.ds(grp * _GRP, _GRP)],
                    rsem[b],
                ).wait()

        def out_slice(c):
            return out_hbm.at[pl.ds((cc0 + c) * _OUT_ROWS, _OUT_ROWS)]

        lane = lax.broadcasted_iota(jnp.int32, (16,), 0)

        def compute(b):
            rows = rowb[b]
            out = outb[b]

            @functools.partial(plsc.parallel_loop, 0, _WPC, unroll=4)
            def _(w):
                base = w * _CTX
                s = [rows[base, pl.ds(k * 16, 16)] for k in range(8)]
                for j in range(_NPAIR):
                    r = base + 1 + j
                    t = [s[k] * rows[r, pl.ds(k * 16, 16)] for k in range(8)]
                    while len(t) > 1:
                        t = [t[i] + t[i + 1] for i in range(0, len(t), 2)]
                    staged[w * _NPAIR + j, :] = t[0]

            @functools.partial(plsc.parallel_loop, 0, _OUT_ROWS // 16)
            def _(g):
                row_idx = g * 16 + lane
                t = [
                    plsc.load_gather(staged, [row_idx, lane * 0 + k])
                    for k in range(16)
                ]
                while len(t) > 1:
                    t = [t[i] + t[i + 1] for i in range(0, len(t), 2)]
                out[pl.ds(g * 16, 16)] = t[0]

        def process(c, b):
            wait_gathers(b)

            @pl.when(c + 2 < _CHUNKS)
            def _(c=c, b=b):
                pltpu.async_copy(ids_hbm.at[cc0 + c + 2], idxb[b], isem[b])

            @pl.when(c >= 2)
            def _(c=c, b=b):
                pltpu.make_async_copy(outb[b], out_slice(c), osem[b]).wait()

            compute(b)
            pltpu.async_copy(outb[b], out_slice(c), osem[b])

            @pl.when(c + 2 < _CHUNKS)
            def _(c=c, b=b):
                pltpu.make_async_copy(
                    ids_hbm.at[cc0 + c + 2], idxb[b], isem[b]
                ).wait()
                start_gathers(b, None)

        pltpu.sync_copy(ids_hbm.at[cc0], idx0)
        pltpu.sync_copy(ids_hbm.at[cc0 + 1], idx1)
        start_gathers(0, None)
        start_gathers(1, None)

        @pl.loop(0, _CHUNKS, step=2)
        def _(c):
            process(c, 0)
            process(c + 1, 1)

        pltpu.make_async_copy(ob0, out_slice(_CHUNKS - 2), osem0).wait()
        pltpu.make_async_copy(ob1, out_slice(_CHUNKS - 1), osem1).wait()

    return sc_kernel(embedding, ids3d)


def _tc_loss_body(pos_ref, neg_ref, out_ref):
    dots_p = pos_ref[...]
    dots_n = neg_ref[...]
    term_p = -jnp.log(jnp.maximum(jax.nn.sigmoid(dots_p), 0.0) + _EPS)
    term_n = -jnp.log(jnp.maximum(1.0 - jax.nn.sigmoid(dots_n), 0.0) + _EPS)
    part = (jnp.sum(term_p) + jnp.sum(term_n)).reshape(1, 1)

    @pl.when(pl.program_id(0) == 0)
    def _():
        out_ref[...] = jnp.zeros((1, 1), jnp.float32)

    out_ref[...] += part


def _tc_loss(dots):
    rows_total = _WALKS * _NPAIR // _D
    half = rows_total // 2
    d2 = dots.reshape(rows_total, _D)
    rb = 720
    nblk = half // rb
    out = pl.pallas_call(
        _tc_loss_body,
        grid=(nblk,),
        in_specs=[
            pl.BlockSpec((rb, _D), lambda i: (i, 0)),
            pl.BlockSpec((rb, _D), lambda i, n=nblk: (i + n, 0)),
        ],
        out_specs=pl.BlockSpec((1, 1), lambda i: (0, 0)),
        out_shape=jax.ShapeDtypeStruct((1, 1), jnp.float32),
    )(d2, d2)
    return out[0, 0]


def kernel(pos_rw, neg_rw, embedding):
    ids = jnp.concatenate(
        [pos_rw.reshape(-1), neg_rw.reshape(-1)]
    ).astype(jnp.int32)
    ids3d = ids.reshape(_NW * _CHUNKS, _NGRP, _GRP)
    dots = _sc_dots(embedding, ids3d)
    total = _tc_loss(dots)
    return total / jnp.float32(_B * _NPAIR)


# speedup vs baseline: 1.0083x; 1.0083x over previous
import dataclasses
import functools

import jax
import jax.numpy as jnp
from jax import lax
from jax.experimental import pallas as pl
from jax.experimental.pallas import tpu as pltpu
from jax.experimental.pallas import tpu_sc as plsc

_NUM_NODES = 100000
_D = 128
_B = 102400
_CTX = 10
_NPAIR = _CTX - 1
_EPS = 1e-15

_NW = 32
_WALKS = 2 * _B
_WPC = 16
_IDS_PER_CHUNK = _WPC * _CTX
_GRP = 80
_NGRP = _IDS_PER_CHUNK // _GRP
_HCHUNKS = _B // (_NW * _WPC)
_OUT_ROWS = _WPC * _NPAIR


def _sc_dots(embedding, pos_ids, neg_ids):
    mesh = plsc.VectorSubcoreMesh(core_axis_name="c", subcore_axis_name="s")
    cp = pltpu.CompilerParams()
    if "needs_layout_passes" in pltpu.CompilerParams.__dataclass_fields__:
        cp = dataclasses.replace(cp, needs_layout_passes=False)

    @functools.partial(
        pl.kernel,
        out_type=jax.ShapeDtypeStruct((_WALKS * _NPAIR,), jnp.float32),
        mesh=mesh,
        compiler_params=cp,
        scratch_types=[
            pltpu.VMEM((_IDS_PER_CHUNK,), jnp.int32),
            pltpu.VMEM((_IDS_PER_CHUNK,), jnp.int32),
            pltpu.VMEM((_IDS_PER_CHUNK, _D), jnp.float32),
            pltpu.VMEM((_IDS_PER_CHUNK, _D), jnp.float32),
            pltpu.VMEM((_OUT_ROWS, 16), jnp.float32),
            pltpu.VMEM((_OUT_ROWS,), jnp.float32),
            pltpu.VMEM((_OUT_ROWS,), jnp.float32),
            pltpu.SemaphoreType.DMA,
            pltpu.SemaphoreType.DMA,
            pltpu.SemaphoreType.DMA,
            pltpu.SemaphoreType.DMA,
            pltpu.SemaphoreType.DMA,
            pltpu.SemaphoreType.DMA,
        ],
    )
    def sc_kernel(table_hbm, pids_hbm, nids_hbm, out_hbm,
                  idx0, idx1, rows0, rows1, staged, ob0, ob1,
                  isem0, isem1, rsem0, rsem1, osem0, osem1):
        wid = lax.axis_index("s") * 2 + lax.axis_index("c")

        idxb = (idx0, idx1)
        rowb = (rows0, rows1)
        outb = (ob0, ob1)
        isem = (isem0, isem1)
        rsem = (rsem0, rsem1)
        osem = (osem0, osem1)

        lane = lax.broadcasted_iota(jnp.int32, (16,), 0)

        def start_gathers(b):
            for grp in range(_NGRP):
                pltpu.async_copy(
                    table_hbm.at[idxb[b].at[pl.ds(grp * _GRP, _GRP)]],
                    rowb[b].at[pl.ds(grp * _GRP, _GRP)],
                    rsem[b],
                )

        def wait_gathers(b):
            for grp in range(_NGRP):
                pltpu.make_async_copy(
                    table_hbm.at[idxb[b].at[pl.ds(grp * _GRP, _GRP)]],
                    rowb[b].at[pl.ds(grp * _GRP, _GRP)],
                    rsem[b],
                ).wait()

        def compute(b):
            rows = rowb[b]
            out = outb[b]

            @functools.partial(plsc.parallel_loop, 0, _WPC, unroll=4)
            def _(w):
                base = w * _CTX
                s = [rows[base, pl.ds(k * 16, 16)] for k in range(8)]
                for j in range(_NPAIR):
                    r = base + 1 + j
                    t = [s[k] * rows[r, pl.ds(k * 16, 16)] for k in range(8)]
                    while len(t) > 1:
                        t = [t[i] + t[i + 1] for i in range(0, len(t), 2)]
                    staged[w * _NPAIR + j, :] = t[0]

            @functools.partial(plsc.parallel_loop, 0, _OUT_ROWS // 16)
            def _(g):
                row_idx = g * 16 + lane
                t = [
                    plsc.load_gather(staged, [row_idx, lane * 0 + k])
                    for k in range(16)
                ]
                while len(t) > 1:
                    t = [t[i] + t[i + 1] for i in range(0, len(t), 2)]
                out[pl.ds(g * 16, 16)] = t[0]

        def run_half(ids_hbm, idbase, outbase):
            def idx_src(c):
                return ids_hbm.at[
                    pl.ds(idbase + c * _IDS_PER_CHUNK, _IDS_PER_CHUNK)
                ]

            def out_dst(c):
                return out_hbm.at[pl.ds(outbase + c * _OUT_ROWS, _OUT_ROWS)]

            def process(c, b):
                wait_gathers(b)

                @pl.when(c + 2 < _HCHUNKS)
                def _(c=c, b=b):
                    pltpu.async_copy(idx_src(c + 2), idxb[b], isem[b])

                @pl.when(c >= 2)
                def _(c=c, b=b):
                    pltpu.make_async_copy(outb[b], out_dst(c), osem[b]).wait()

                compute(b)
                pltpu.async_copy(outb[b], out_dst(c), osem[b])

                @pl.when(c + 2 < _HCHUNKS)
                def _(c=c, b=b):
                    pltpu.make_async_copy(idx_src(c + 2), idxb[b], isem[b]).wait()
                    start_gathers(b)

            pltpu.sync_copy(idx_src(0), idx0)
            pltpu.sync_copy(idx_src(1), idx1)
            start_gathers(0)
            start_gathers(1)

            @pl.loop(0, _HCHUNKS, step=2)
            def _(c):
                process(c, 0)
                process(c + 1, 1)

            pltpu.make_async_copy(ob0, out_dst(_HCHUNKS - 2), osem0).wait()
            pltpu.make_async_copy(ob1, out_dst(_HCHUNKS - 1), osem1).wait()

        walks_per_sub = _HCHUNKS * _WPC
        run_half(pids_hbm, wid * walks_per_sub * _CTX,
                 wid * walks_per_sub * _NPAIR)
        run_half(nids_hbm, wid * walks_per_sub * _CTX,
                 _B * _NPAIR + wid * walks_per_sub * _NPAIR)

    return sc_kernel(embedding, pos_ids, neg_ids)


def _tc_loss_body(pos_ref, neg_ref, out_ref):
    dots_p = pos_ref[...]
    dots_n = neg_ref[...]
    term_p = -jnp.log(jnp.maximum(jax.nn.sigmoid(dots_p), 0.0) + _EPS)
    term_n = -jnp.log(jnp.maximum(1.0 - jax.nn.sigmoid(dots_n), 0.0) + _EPS)
    part = (jnp.sum(term_p) + jnp.sum(term_n)).reshape(1, 1)

    @pl.when(pl.program_id(0) == 0)
    def _():
        out_ref[...] = jnp.zeros((1, 1), jnp.float32)

    out_ref[...] += part


def _tc_loss(dots):
    rows_total = _WALKS * _NPAIR // _D
    half = rows_total // 2
    d2 = dots.reshape(rows_total, _D)
    rb = 720
    nblk = half // rb
    out = pl.pallas_call(
        _tc_loss_body,
        grid=(nblk,),
        in_specs=[
            pl.BlockSpec((rb, _D), lambda i: (i, 0)),
            pl.BlockSpec((rb, _D), lambda i, n=nblk: (i + n, 0)),
        ],
        out_specs=pl.BlockSpec((1, 1), lambda i: (0, 0)),
        out_shape=jax.ShapeDtypeStruct((1, 1), jnp.float32),
    )(d2, d2)
    return out[0, 0]


def kernel(pos_rw, neg_rw, embedding):
    pos_ids = pos_rw.reshape(-1).astype(jnp.int32)
    neg_ids = neg_rw.reshape(-1).astype(jnp.int32)
    dots = _sc_dots(embedding, pos_ids, neg_ids)
    total = _tc_loss(dots)
    return total / jnp.float32(_B * _NPAIR)


# speedup vs baseline: 1.0997x; 1.0907x over previous
import dataclasses
import functools

import jax
import jax.numpy as jnp
from jax import lax
from jax.experimental import pallas as pl
from jax.experimental.pallas import tpu as pltpu
from jax.experimental.pallas import tpu_sc as plsc

_NUM_NODES = 100000
_D = 128
_B = 102400
_CTX = 10
_NPAIR = _CTX - 1
_EPS = 1e-15

_NW = 32
_WALKS = 2 * _B
_WPC = 16
_IDS_PER_CHUNK = _WPC * _CTX
_GRP = 80
_NGRP = _IDS_PER_CHUNK // _GRP
_HCHUNKS = _B // (_NW * _WPC)
_OUT_ROWS = _WPC * _NPAIR


def _sc_dots(embedding, pos_ids, neg_ids):
    mesh = plsc.VectorSubcoreMesh(core_axis_name="c", subcore_axis_name="s")
    cp = pltpu.CompilerParams()
    if "needs_layout_passes" in pltpu.CompilerParams.__dataclass_fields__:
        cp = dataclasses.replace(cp, needs_layout_passes=False)

    @functools.partial(
        pl.kernel,
        out_type=jax.ShapeDtypeStruct((_WALKS * _NPAIR,), jnp.float32),
        mesh=mesh,
        compiler_params=cp,
        scratch_types=[
            pltpu.VMEM((_WPC, _CTX), jnp.int32),
            pltpu.VMEM((_WPC, _CTX), jnp.int32),
            pltpu.VMEM((_IDS_PER_CHUNK,), jnp.int32),
            pltpu.VMEM((_IDS_PER_CHUNK,), jnp.int32),
            pltpu.VMEM((_IDS_PER_CHUNK, _D), jnp.float32),
            pltpu.VMEM((_IDS_PER_CHUNK, _D), jnp.float32),
            pltpu.VMEM((_OUT_ROWS, 16), jnp.float32),
            pltpu.VMEM((_OUT_ROWS,), jnp.float32),
            pltpu.VMEM((_OUT_ROWS,), jnp.float32),
            pltpu.SemaphoreType.DMA,
            pltpu.SemaphoreType.DMA,
            pltpu.SemaphoreType.DMA,
            pltpu.SemaphoreType.DMA,
            pltpu.SemaphoreType.DMA,
            pltpu.SemaphoreType.DMA,
        ],
    )
    def sc_kernel(table_hbm, pids_hbm, nids_hbm, out_hbm,
                  raw0, raw1, idx0, idx1, rows0, rows1, staged, ob0, ob1,
                  isem0, isem1, rsem0, rsem1, osem0, osem1):
        wid = lax.axis_index("s") * 2 + lax.axis_index("c")

        rawb = (raw0, raw1)
        idxb = (idx0, idx1)
        rowb = (rows0, rows1)
        outb = (ob0, ob1)
        isem = (isem0, isem1)
        rsem = (rsem0, rsem1)
        osem = (osem0, osem1)

        lane = lax.broadcasted_iota(jnp.int32, (16,), 0)

        def flatten_ids(b):
            for g in range(_IDS_PER_CHUNK // 16):
                p = g * 16 + lane
                row = (p * 6554) >> 16
                col = p - row * _CTX
                idxb[b][pl.ds(g * 16, 16)] = plsc.load_gather(
                    rawb[b], [row, col]
                )

        def start_gathers(b):
            for grp in range(_NGRP):
                pltpu.async_copy(
                    table_hbm.at[idxb[b].at[pl.ds(grp * _GRP, _GRP)]],
                    rowb[b].at[pl.ds(grp * _GRP, _GRP)],
                    rsem[b],
                )

        def wait_gathers(b):
            for grp in range(_NGRP):
                pltpu.make_async_copy(
                    table_hbm.at[idxb[b].at[pl.ds(grp * _GRP, _GRP)]],
                    rowb[b].at[pl.ds(grp * _GRP, _GRP)],
                    rsem[b],
                ).wait()

        def compute(b):
            rows = rowb[b]
            out = outb[b]

            @functools.partial(plsc.parallel_loop, 0, _WPC, unroll=4)
            def _(w):
                base = w * _CTX
                s = [rows[base, pl.ds(k * 16, 16)] for k in range(8)]
                for j in range(_NPAIR):
                    r = base + 1 + j
                    t = [s[k] * rows[r, pl.ds(k * 16, 16)] for k in range(8)]
                    while len(t) > 1:
                        t = [t[i] + t[i + 1] for i in range(0, len(t), 2)]
                    staged[w * _NPAIR + j, :] = t[0]

            @functools.partial(plsc.parallel_loop, 0, _OUT_ROWS // 16)
            def _(g):
                row_idx = g * 16 + lane
                t = [
                    plsc.load_gather(staged, [row_idx, lane * 0 + k])
                    for k in range(16)
                ]
                while len(t) > 1:
                    t = [t[i] + t[i + 1] for i in range(0, len(t), 2)]
                out[pl.ds(g * 16, 16)] = t[0]

        def run_half(ids_hbm, widbase, outbase):
            def idx_src(c):
                return ids_hbm.at[pl.ds(widbase + c * _WPC, _WPC), :]

            def out_dst(c):
                return out_hbm.at[pl.ds(outbase + c * _OUT_ROWS, _OUT_ROWS)]

            def process(c, b):
                wait_gathers(b)

                @pl.when(c + 2 < _HCHUNKS)
                def _(c=c, b=b):
                    pltpu.async_copy(idx_src(c + 2), rawb[b], isem[b])

                @pl.when(c >= 2)
                def _(c=c, b=b):
                    pltpu.make_async_copy(outb[b], out_dst(c), osem[b]).wait()

                compute(b)
                pltpu.async_copy(outb[b], out_dst(c), osem[b])

                @pl.when(c + 2 < _HCHUNKS)
                def _(c=c, b=b):
                    pltpu.make_async_copy(idx_src(c + 2), rawb[b], isem[b]).wait()
                    flatten_ids(b)
                    start_gathers(b)

            pltpu.sync_copy(idx_src(0), raw0)
            flatten_ids(0)
            pltpu.sync_copy(idx_src(1), raw1)
            flatten_ids(1)
            start_gathers(0)
            start_gathers(1)

            @pl.loop(0, _HCHUNKS, step=2)
            def _(c):
                process(c, 0)
                process(c + 1, 1)

            pltpu.make_async_copy(ob0, out_dst(_HCHUNKS - 2), osem0).wait()
            pltpu.make_async_copy(ob1, out_dst(_HCHUNKS - 1), osem1).wait()

        walks_per_sub = _HCHUNKS * _WPC
        run_half(pids_hbm, wid * walks_per_sub,
                 wid * walks_per_sub * _NPAIR)
        run_half(nids_hbm, wid * walks_per_sub,
                 _B * _NPAIR + wid * walks_per_sub * _NPAIR)

    return sc_kernel(embedding, pos_ids, neg_ids)


def _tc_loss_body(pos_ref, neg_ref, out_ref):
    dots_p = pos_ref[...]
    dots_n = neg_ref[...]
    term_p = -jnp.log(jnp.maximum(jax.nn.sigmoid(dots_p), 0.0) + _EPS)
    term_n = -jnp.log(jnp.maximum(1.0 - jax.nn.sigmoid(dots_n), 0.0) + _EPS)
    part = (jnp.sum(term_p) + jnp.sum(term_n)).reshape(1, 1)

    @pl.when(pl.program_id(0) == 0)
    def _():
        out_ref[...] = jnp.zeros((1, 1), jnp.float32)

    out_ref[...] += part


def _tc_loss(dots):
    rows_total = _WALKS * _NPAIR // _D
    half = rows_total // 2
    d2 = dots.reshape(rows_total, _D)
    rb = 720
    nblk = half // rb
    out = pl.pallas_call(
        _tc_loss_body,
        grid=(nblk,),
        in_specs=[
            pl.BlockSpec((rb, _D), lambda i: (i, 0)),
            pl.BlockSpec((rb, _D), lambda i, n=nblk: (i + n, 0)),
        ],
        out_specs=pl.BlockSpec((1, 1), lambda i: (0, 0)),
        out_shape=jax.ShapeDtypeStruct((1, 1), jnp.float32),
    )(d2, d2)
    return out[0, 0]


def kernel(pos_rw, neg_rw, embedding):
    pos_ids = pos_rw.astype(jnp.int32)
    neg_ids = neg_rw.astype(jnp.int32)
    dots = _sc_dots(embedding, pos_ids, neg_ids)
    total = _tc_loss(dots)
    return total / jnp.float32(_B * _NPAIR)
